# baseline (speedup 1.0000x reference)
import functools

import jax
import jax.numpy as jnp
from jax import lax
from jax.experimental import pallas as pl
from jax.experimental.pallas import tpu as pltpu
from jax.experimental.pallas import tpu_sc as plsc

N = 10000
E = 320000
D = 128
DE = 16
H = 8
DH = 16

NC = 2
NS = 16
NW = NC * NS
PER_TILE = E // NW
C = 32
RG = 16
GC = 4
W = 144
NPAD = 10240
ROWS_PER_TILE = NPAD // NS


def _qkv_body(x_ref, wq_ref, wk_ref, wv_ref, q_ref, kv_ref):
    xb = x_ref[...]
    q_ref[...] = jnp.dot(xb, wq_ref[...], preferred_element_type=jnp.float32)
    kv_ref[:, :D] = jnp.dot(xb, wk_ref[...], preferred_element_type=jnp.float32)
    kv_ref[:, D:] = jnp.dot(xb, wv_ref[...], preferred_element_type=jnp.float32)


def _qkv(x, WQ, WK, WV):
    blk = 1000
    grid = (N // blk,)
    spec_x = pl.BlockSpec((blk, D), lambda i: (i, 0))
    spec_w = pl.BlockSpec((D, D), lambda i: (0, 0))
    return pl.pallas_call(
        _qkv_body,
        grid=grid,
        in_specs=[spec_x, spec_w, spec_w, spec_w],
        out_specs=[pl.BlockSpec((blk, D), lambda i: (i, 0)),
                   pl.BlockSpec((blk, 2 * D), lambda i: (i, 0))],
        out_shape=[jax.ShapeDtypeStruct((N, D), jnp.float32),
                   jax.ShapeDtypeStruct((N, 2 * D), jnp.float32)],
    )(x, WQ, WK, WV)


def _eproj_body(ea_ref, we_ref, eh_ref):
    eh_ref[...] = jnp.dot(ea_ref[...], we_ref[...],
                          preferred_element_type=jnp.float32)


def _eproj(edge_attr, WE):
    blk = 4000
    grid = (E // blk,)
    return pl.pallas_call(
        _eproj_body,
        grid=grid,
        in_specs=[pl.BlockSpec((blk, DE), lambda i: (i, 0)),
                  pl.BlockSpec((DE, D), lambda i: (0, 0))],
        out_specs=pl.BlockSpec((blk, D), lambda i: (i, 0)),
        out_shape=jax.ShapeDtypeStruct((E, D), jnp.float32),
    )(edge_attr, WE)


_mesh = plsc.VectorSubcoreMesh(core_axis_name="c", subcore_axis_name="s")

_GDN = lax.GatherDimensionNumbers(
    offset_dims=(), collapsed_slice_dims=(0,), start_index_map=(0,))


def _shuf(v, perm):
    return lax.gather(v, perm[:, None], _GDN, (1,),
                      mode=lax.GatherScatterMode.PROMISE_IN_BOUNDS)


NF = PER_TILE // C
CT = PER_TILE - NF * C


@functools.partial(
    pl.kernel,
    out_type=jax.ShapeDtypeStruct((NC * NPAD, W), jnp.float32),
    mesh=_mesh,
    scratch_types=[
        pltpu.VMEM((2 * GC, RG), jnp.int32),
        pltpu.VMEM((2 * GC, RG), jnp.int32),
        pltpu.VMEM((CT,), jnp.int32),
        pltpu.VMEM((CT,), jnp.int32),
        pltpu.VMEM((C, 2 * D), jnp.float32),
        pltpu.VMEM((C, 2 * D), jnp.float32),
        pltpu.VMEM((C, D), jnp.float32),
        pltpu.VMEM((C, D), jnp.float32),
        pltpu.VMEM((C, D), jnp.float32),
        pltpu.VMEM((C, D), jnp.float32),
        pltpu.VMEM((C, W), jnp.float32),
        pltpu.VMEM_SHARED((NPAD, W), jnp.float32),
        pltpu.SemaphoreType.DMA,
        pltpu.SemaphoreType.DMA,
    ],
    compiler_params=pltpu.CompilerParams(use_tc_tiling_on_sc=False),
)
def _sc_attn(kvh, qh, eh, src16, dst16, out,
             srcbig, dstbig, srcT, dstT,
             kvA, kvB, qA, qB, eA, eB, msgbuf, acc, gsemA, gsemB):
    c = lax.axis_index("c")
    s = lax.axis_index("s")
    lane = lax.iota(jnp.int32, 16)
    zero16 = jnp.zeros((16,), jnp.float32)

    def zrow(r, carry):
        for j in range(W // 16):
            msgbuf[r, pl.ds(j * 16, 16)] = zero16
        return carry
    lax.fori_loop(0, C, zrow, 0)
    row0 = s * ROWS_PER_TILE
    for i in range(ROWS_PER_TILE // C):
        pltpu.sync_copy(msgbuf, acc.at[pl.ds(row0 + i * C, C)])
    plsc.subcore_barrier()

    tile_base = (c * NS + s) * PER_TILE

    def merge(a, b, k):
        m = (lane & k) == 0
        pk = lane ^ k
        return (jnp.where(m, a, b) + jnp.where(m, _shuf(a, pk),
                                               _shuf(b, pk)))

    def do_chunk(kvb, qb, eb, dstb, count):
        n_edges = C if count == -1 else count
        @plsc.parallel_loop(0, n_edges, 1, unroll=2)
        def edge_body(e):
            t = [kvb[e, pl.ds(h * DH, DH)]
                 * qb[e, pl.ds(h * DH, DH)]
                 * eb[e, pl.ds(h * DH, DH)] for h in range(H)]
            m = [merge(t[2 * j], t[2 * j + 1], 1) for j in range(4)]
            n = [merge(m[2 * j], m[2 * j + 1], 2) for j in range(2)]
            p = merge(n[0], n[1], 4)
            sacc = p + _shuf(p, lane ^ 8)
            score = jnp.exp(jnp.clip(sacc * 0.25, -5.0, 5.0))
            msgbuf[e, pl.ds(D, 16)] = score
            for h in range(H):
                bc = _shuf(score, jnp.full((16,), h, jnp.int32))
                msgbuf[e, pl.ds(h * DH, DH)] = (
                    kvb[e, pl.ds(D + h * DH, DH)] * bc)
        if count == -1:
            return
        pltpu.sync_copy(msgbuf.at[pl.ds(0, count)], acc.at[dstb], add=True)

    def issue(g, k, kvb, qb, eb, sem):
        pltpu.async_copy(kvh.at[srcbig.at[2 * k]], kvb.at[pl.ds(0, RG)], sem)
        pltpu.async_copy(kvh.at[srcbig.at[2 * k + 1]],
                         kvb.at[pl.ds(RG, RG)], sem)
        pltpu.async_copy(qh.at[dstbig.at[2 * k]], qb.at[pl.ds(0, RG)], sem)
        pltpu.async_copy(qh.at[dstbig.at[2 * k + 1]],
                         qb.at[pl.ds(RG, RG)], sem)
        ebase = tile_base + (g * GC + k) * C
        pltpu.async_copy(eh.at[pl.ds(ebase, C)], eb, sem)

    def drain(kvb, qb, eb, sem):
        for o in (0, RG):
            pltpu.make_async_copy(kvh.at[pl.ds(0, RG)],
                                  kvb.at[pl.ds(o, RG)], sem).wait()
            pltpu.make_async_copy(qh.at[pl.ds(0, RG)],
                                  qb.at[pl.ds(o, RG)], sem).wait()
        pltpu.make_async_copy(eh.at[pl.ds(0, C)], eb, sem).wait()

    bufs = [(kvA, qA, eA, gsemA), (kvB, qB, eB, gsemB)]
    idx_row0 = (c * NS + s) * (PER_TILE // RG)

    def group_body(g, carry):
        r0 = idx_row0 + g * (2 * GC)
        pltpu.sync_copy(src16.at[pl.ds(r0, 2 * GC)], srcbig)
        pltpu.sync_copy(dst16.at[pl.ds(r0, 2 * GC)], dstbig)
        issue(g, 0, *bufs[0])
        for k in range(GC):
            if k + 1 < GC:
                issue(g, k + 1, *bufs[(k + 1) % 2])
            kvb, qb, eb, sem = bufs[k % 2]
            drain(kvb, qb, eb, sem)
            do_chunk(kvb, qb, eb, None, -1)
            pltpu.sync_copy(msgbuf.at[pl.ds(0, RG)],
                            acc.at[dstbig.at[2 * k]], add=True)
            pltpu.sync_copy(msgbuf.at[pl.ds(RG, RG)],
                            acc.at[dstbig.at[2 * k + 1]], add=True)
        return carry
    lax.fori_loop(0, NF // GC, group_body, 0)

    trow = idx_row0 + PER_TILE // RG - 1
    pltpu.sync_copy(src16.at[trow], srcT)
    pltpu.sync_copy(dst16.at[trow], dstT)
    tbase = tile_base + NF * C
    pltpu.async_copy(kvh.at[srcT], kvA.at[pl.ds(0, CT)], gsemA)
    pltpu.async_copy(qh.at[dstT], qA.at[pl.ds(0, CT)], gsemA)
    pltpu.async_copy(eh.at[pl.ds(tbase, CT)], eA.at[pl.ds(0, CT)], gsemA)
    pltpu.make_async_copy(kvh.at[pl.ds(0, CT)], kvA.at[pl.ds(0, CT)],
                          gsemA).wait()
    pltpu.make_async_copy(qh.at[pl.ds(0, CT)], qA.at[pl.ds(0, CT)],
                          gsemA).wait()
    pltpu.make_async_copy(eh.at[pl.ds(0, CT)], eA.at[pl.ds(0, CT)],
                          gsemA).wait()
    do_chunk(kvA, qA, eA, dstT, CT)

    plsc.subcore_barrier()
    out_base = c * NPAD + row0
    pltpu.sync_copy(acc.at[pl.ds(row0, ROWS_PER_TILE)],
                    out.at[pl.ds(out_base, ROWS_PER_TILE)])


def _combine_body(p0_ref, p1_ref, o_ref):
    a = p0_ref[0] + p1_ref[0]
    for h in range(H):
        wv = a[:, h * DH:(h + 1) * DH]
        z = a[:, D + h:D + h + 1]
        o_ref[:, h * DH:(h + 1) * DH] = wv / (z + 1e-6)


def _combine(partials):
    blk = 1000
    grid = (N // blk,)
    p3 = partials.reshape(NC, NPAD, W)
    return pl.pallas_call(
        _combine_body,
        grid=grid,
        in_specs=[pl.BlockSpec((1, blk, W), lambda i: (0, i, 0)),
                  pl.BlockSpec((1, blk, W), lambda i: (1, i, 0))],
        out_specs=pl.BlockSpec((blk, D), lambda i: (i, 0)),
        out_shape=jax.ShapeDtypeStruct((N, D), jnp.float32),
    )(p3, p3)


def kernel(x, edge_index, edge_attr, WQ, WK, WE, WV):
    qh, kvh = _qkv(x, WQ, WK, WV)
    eh = _eproj(edge_attr, WE)
    idx16 = edge_index.reshape(2, E // RG, RG)
    partials = _sc_attn(kvh, qh, eh, idx16[0], idx16[1])
    return _combine(partials)


# speedup vs baseline: 1.1019x; 1.1019x over previous
import functools

import jax
import jax.numpy as jnp
from jax import lax
from jax.experimental import pallas as pl
from jax.experimental.pallas import tpu as pltpu
from jax.experimental.pallas import tpu_sc as plsc

N = 10000
E = 320000
D = 128
DE = 16
H = 8
DH = 16

NC = 2
NS = 16
NW = NC * NS
PER_TILE = E // NW
C = 32
W = 144
NPAD = 10240
ROWS_PER_TILE = NPAD // NS


def _qkv_body(x_ref, wq_ref, wk_ref, wv_ref, q_ref, kv_ref):
    xb = x_ref[...]
    q_ref[...] = jnp.dot(xb, wq_ref[...], preferred_element_type=jnp.float32)
    kv_ref[:, :D] = jnp.dot(xb, wk_ref[...], preferred_element_type=jnp.float32)
    kv_ref[:, D:] = jnp.dot(xb, wv_ref[...], preferred_element_type=jnp.float32)


def _qkv(x, WQ, WK, WV):
    blk = 1000
    grid = (N // blk,)
    spec_x = pl.BlockSpec((blk, D), lambda i: (i, 0))
    spec_w = pl.BlockSpec((D, D), lambda i: (0, 0))
    return pl.pallas_call(
        _qkv_body,
        grid=grid,
        in_specs=[spec_x, spec_w, spec_w, spec_w],
        out_specs=[pl.BlockSpec((blk, D), lambda i: (i, 0)),
                   pl.BlockSpec((blk, 2 * D), lambda i: (i, 0))],
        out_shape=[jax.ShapeDtypeStruct((N, D), jnp.float32),
                   jax.ShapeDtypeStruct((N, 2 * D), jnp.float32)],
    )(x, WQ, WK, WV)


def _eproj_body(ea_ref, we_ref, eh_ref):
    eh_ref[...] = jnp.dot(ea_ref[...], we_ref[...],
                          preferred_element_type=jnp.float32)


def _eproj(edge_attr, WE):
    blk = 4000
    grid = (E // blk,)
    return pl.pallas_call(
        _eproj_body,
        grid=grid,
        in_specs=[pl.BlockSpec((blk, DE), lambda i: (i, 0)),
                  pl.BlockSpec((DE, D), lambda i: (0, 0))],
        out_specs=pl.BlockSpec((blk, D), lambda i: (i, 0)),
        out_shape=jax.ShapeDtypeStruct((E, D), jnp.float32),
    )(edge_attr, WE)


_mesh = plsc.VectorSubcoreMesh(core_axis_name="c", subcore_axis_name="s")

_GDN = lax.GatherDimensionNumbers(
    offset_dims=(), collapsed_slice_dims=(0,), start_index_map=(0,))


def _shuf(v, perm):
    return lax.gather(v, perm[:, None], _GDN, (1,),
                      mode=lax.GatherScatterMode.PROMISE_IN_BOUNDS)


NF = PER_TILE // C
CT = PER_TILE - NF * C


@functools.partial(
    pl.kernel,
    out_type=jax.ShapeDtypeStruct((NC * NPAD, W), jnp.float32),
    mesh=_mesh,
    scratch_types=[
        pltpu.VMEM((C,), jnp.int32),
        pltpu.VMEM((C,), jnp.int32),
        pltpu.VMEM((C,), jnp.int32),
        pltpu.VMEM((C,), jnp.int32),
        pltpu.VMEM((CT,), jnp.int32),
        pltpu.VMEM((CT,), jnp.int32),
        pltpu.VMEM((C, 2 * D), jnp.float32),
        pltpu.VMEM((C, 2 * D), jnp.float32),
        pltpu.VMEM((C, D), jnp.float32),
        pltpu.VMEM((C, D), jnp.float32),
        pltpu.VMEM((C, D), jnp.float32),
        pltpu.VMEM((C, D), jnp.float32),
        pltpu.VMEM((C, W), jnp.float32),
        pltpu.VMEM_SHARED((NPAD, W), jnp.float32),
        pltpu.SemaphoreType.DMA,
        pltpu.SemaphoreType.DMA,
    ],
    compiler_params=pltpu.CompilerParams(use_tc_tiling_on_sc=False),
)
def _sc_attn(kvh, qh, eh, eidx, out,
             srcA, dstA, srcB, dstB, srcT, dstT,
             kvA, kvB, qA, qB, eA, eB, msgbuf, acc, gsemA, gsemB):
    c = lax.axis_index("c")
    s = lax.axis_index("s")
    lane = lax.iota(jnp.int32, 16)
    zero16 = jnp.zeros((16,), jnp.float32)

    def zrow(r, carry):
        for j in range(W // 16):
            msgbuf[r, pl.ds(j * 16, 16)] = zero16
        return carry
    lax.fori_loop(0, C, zrow, 0)
    row0 = s * ROWS_PER_TILE
    for i in range(ROWS_PER_TILE // C):
        pltpu.sync_copy(msgbuf, acc.at[pl.ds(row0 + i * C, C)])
    plsc.subcore_barrier()

    tile_base = (c * NS + s) * PER_TILE

    def merge(a, b, k):
        m = (lane & k) == 0
        pk = lane ^ k
        return (jnp.where(m, a, b) + jnp.where(m, _shuf(a, pk),
                                               _shuf(b, pk)))

    def do_chunk(kvb, qb, eb, dstb, count):
        @plsc.parallel_loop(0, count, 1, unroll=2)
        def edge_body(e):
            t = [kvb[e, pl.ds(h * DH, DH)]
                 * qb[e, pl.ds(h * DH, DH)]
                 * eb[e, pl.ds(h * DH, DH)] for h in range(H)]
            m = [merge(t[2 * j], t[2 * j + 1], 1) for j in range(4)]
            n = [merge(m[2 * j], m[2 * j + 1], 2) for j in range(2)]
            p = merge(n[0], n[1], 4)
            sacc = p + _shuf(p, lane ^ 8)
            score = jnp.exp(jnp.clip(sacc * 0.25, -5.0, 5.0))
            msgbuf[e, pl.ds(D, 16)] = score
            for h in range(H):
                bc = _shuf(score, jnp.full((16,), h, jnp.int32))
                msgbuf[e, pl.ds(h * DH, DH)] = (
                    kvb[e, pl.ds(D + h * DH, DH)] * bc)
        if count == C:
            pltpu.sync_copy(msgbuf, acc.at[dstb], add=True)
        else:
            pltpu.sync_copy(msgbuf.at[pl.ds(0, count)], acc.at[dstb],
                            add=True)

    def load_idx(base, srcb, dstb):
        pltpu.sync_copy(eidx.at[0, pl.ds(base, srcb.shape[0])], srcb)
        pltpu.sync_copy(eidx.at[1, pl.ds(base, dstb.shape[0])], dstb)

    def issue(base, srcb, dstb, kvb, qb, eb, sem):
        pltpu.async_copy(kvh.at[srcb], kvb, sem)
        pltpu.async_copy(qh.at[dstb], qb, sem)
        pltpu.async_copy(eh.at[pl.ds(base, kvb.shape[0])], eb, sem)

    def drain(kvb, qb, eb, sem):
        pltpu.make_async_copy(kvh.at[pl.ds(0, kvb.shape[0])], kvb,
                              sem).wait()
        pltpu.make_async_copy(qh.at[pl.ds(0, qb.shape[0])], qb, sem).wait()
        pltpu.make_async_copy(eh.at[pl.ds(0, eb.shape[0])], eb, sem).wait()

    load_idx(tile_base, srcA, dstA)
    issue(tile_base, srcA, dstA, kvA, qA, eA, gsemA)

    def pair_body(j, carry):
        i1 = 2 * j + 1
        base1 = tile_base + i1 * C
        load_idx(base1, srcB, dstB)
        issue(base1, srcB, dstB, kvB, qB, eB, gsemB)
        drain(kvA, qA, eA, gsemA)
        do_chunk(kvA, qA, eA, dstA, C)
        i2 = jnp.minimum(2 * j + 2, NF - 1)
        base2 = tile_base + i2 * C
        load_idx(base2, srcA, dstA)
        issue(base2, srcA, dstA, kvA, qA, eA, gsemA)
        drain(kvB, qB, eB, gsemB)
        do_chunk(kvB, qB, eB, dstB, C)
        return carry
    lax.fori_loop(0, NF // 2, pair_body, 0)

    drain(kvA, qA, eA, gsemA)

    tbase = tile_base + NF * C
    pltpu.sync_copy(eidx.at[0, pl.ds(tbase, CT)], srcT)
    pltpu.sync_copy(eidx.at[1, pl.ds(tbase, CT)], dstT)
    pltpu.async_copy(kvh.at[srcT], kvA.at[pl.ds(0, CT)], gsemA)
    pltpu.async_copy(qh.at[dstT], qA.at[pl.ds(0, CT)], gsemA)
    pltpu.async_copy(eh.at[pl.ds(tbase, CT)], eA.at[pl.ds(0, CT)], gsemA)
    pltpu.make_async_copy(kvh.at[pl.ds(0, CT)], kvA.at[pl.ds(0, CT)],
                          gsemA).wait()
    pltpu.make_async_copy(qh.at[pl.ds(0, CT)], qA.at[pl.ds(0, CT)],
                          gsemA).wait()
    pltpu.make_async_copy(eh.at[pl.ds(0, CT)], eA.at[pl.ds(0, CT)],
                          gsemA).wait()
    do_chunk(kvA, qA, eA, dstT, CT)

    plsc.subcore_barrier()
    out_base = c * NPAD + row0
    pltpu.sync_copy(acc.at[pl.ds(row0, ROWS_PER_TILE)],
                    out.at[pl.ds(out_base, ROWS_PER_TILE)])


def _combine_body(p0_ref, p1_ref, o_ref):
    a = p0_ref[0] + p1_ref[0]
    for h in range(H):
        wv = a[:, h * DH:(h + 1) * DH]
        z = a[:, D + h:D + h + 1]
        o_ref[:, h * DH:(h + 1) * DH] = wv / (z + 1e-6)


def _combine(partials):
    blk = 1000
    grid = (N // blk,)
    p3 = partials.reshape(NC, NPAD, W)
    return pl.pallas_call(
        _combine_body,
        grid=grid,
        in_specs=[pl.BlockSpec((1, blk, W), lambda i: (0, i, 0)),
                  pl.BlockSpec((1, blk, W), lambda i: (1, i, 0))],
        out_specs=pl.BlockSpec((blk, D), lambda i: (i, 0)),
        out_shape=jax.ShapeDtypeStruct((N, D), jnp.float32),
    )(p3, p3)


def kernel(x, edge_index, edge_attr, WQ, WK, WE, WV):
    qh, kvh = _qkv(x, WQ, WK, WV)
    eh = _eproj(edge_attr, WE)
    partials = _sc_attn(kvh, qh, eh, edge_index)
    return _combine(partials)


# speedup vs baseline: 1.2209x; 1.1080x over previous
import functools

import jax
import jax.numpy as jnp
from jax import lax
from jax.experimental import pallas as pl
from jax.experimental.pallas import tpu as pltpu
from jax.experimental.pallas import tpu_sc as plsc

N = 10000
E = 320000
D = 128
DE = 16
H = 8
DH = 16

NC = 2
NS = 16
NW = NC * NS
PER_TILE = E // NW
C = 32
W = 144
NPAD = 10240
ROWS_PER_TILE = NPAD // NS


def _qkv_body(x_ref, wq_ref, wk_ref, wv_ref, q_ref, kv_ref):
    xb = x_ref[...]
    q_ref[...] = jnp.dot(xb, wq_ref[...], preferred_element_type=jnp.float32)
    kv_ref[:, :D] = jnp.dot(xb, wk_ref[...], preferred_element_type=jnp.float32)
    kv_ref[:, D:] = jnp.dot(xb, wv_ref[...], preferred_element_type=jnp.float32)


def _qkv(x, WQ, WK, WV):
    blk = 1000
    grid = (N // blk,)
    spec_x = pl.BlockSpec((blk, D), lambda i: (i, 0))
    spec_w = pl.BlockSpec((D, D), lambda i: (0, 0))
    return pl.pallas_call(
        _qkv_body,
        grid=grid,
        in_specs=[spec_x, spec_w, spec_w, spec_w],
        out_specs=[pl.BlockSpec((blk, D), lambda i: (i, 0)),
                   pl.BlockSpec((blk, 2 * D), lambda i: (i, 0))],
        out_shape=[jax.ShapeDtypeStruct((N, D), jnp.float32),
                   jax.ShapeDtypeStruct((N, 2 * D), jnp.float32)],
    )(x, WQ, WK, WV)


def _eproj_body(ea_ref, we_ref, eh_ref):
    eh_ref[...] = jnp.dot(ea_ref[...], we_ref[...],
                          preferred_element_type=jnp.float32)


def _eproj(edge_attr, WE):
    blk = 4000
    grid = (E // blk,)
    return pl.pallas_call(
        _eproj_body,
        grid=grid,
        in_specs=[pl.BlockSpec((blk, DE), lambda i: (i, 0)),
                  pl.BlockSpec((DE, D), lambda i: (0, 0))],
        out_specs=pl.BlockSpec((blk, D), lambda i: (i, 0)),
        out_shape=jax.ShapeDtypeStruct((E, D), jnp.float32),
    )(edge_attr, WE)


_mesh = plsc.VectorSubcoreMesh(core_axis_name="c", subcore_axis_name="s")

_GDN = lax.GatherDimensionNumbers(
    offset_dims=(), collapsed_slice_dims=(0,), start_index_map=(0,))


def _shuf(v, perm):
    return lax.gather(v, perm[:, None], _GDN, (1,),
                      mode=lax.GatherScatterMode.PROMISE_IN_BOUNDS)


NF = PER_TILE // C
CT = PER_TILE - NF * C


@functools.partial(
    pl.kernel,
    out_type=jax.ShapeDtypeStruct((NC * NPAD, W), jnp.float32),
    mesh=_mesh,
    scratch_types=[
        pltpu.VMEM((2, C), jnp.int32),
        pltpu.VMEM((2, C), jnp.int32),
        pltpu.VMEM((2, CT), jnp.int32),
        pltpu.VMEM((C, 2 * D), jnp.float32),
        pltpu.VMEM((C, 2 * D), jnp.float32),
        pltpu.VMEM((C, D), jnp.float32),
        pltpu.VMEM((C, D), jnp.float32),
        pltpu.VMEM((C, D), jnp.float32),
        pltpu.VMEM((C, D), jnp.float32),
        pltpu.VMEM((C, W), jnp.float32),
        pltpu.VMEM_SHARED((NPAD, W), jnp.float32),
        pltpu.SemaphoreType.DMA,
        pltpu.SemaphoreType.DMA,
    ],
    compiler_params=pltpu.CompilerParams(use_tc_tiling_on_sc=False),
)
def _sc_attn(kvh, qh, eh, eidx, out,
             idxA, idxB, idxT,
             kvA, kvB, qA, qB, eA, eB, msgbuf, acc, gsemA, gsemB):
    c = lax.axis_index("c")
    s = lax.axis_index("s")
    lane = lax.iota(jnp.int32, 16)
    zero16 = jnp.zeros((16,), jnp.float32)

    def zrow(r, carry):
        for j in range(W // 16):
            msgbuf[r, pl.ds(j * 16, 16)] = zero16
        return carry
    lax.fori_loop(0, C, zrow, 0)
    row0 = s * ROWS_PER_TILE
    for i in range(ROWS_PER_TILE // C):
        pltpu.sync_copy(msgbuf, acc.at[pl.ds(row0 + i * C, C)])
    plsc.subcore_barrier()

    tile_base = (c * NS + s) * PER_TILE

    def merge(a, b, k):
        m = (lane & k) == 0
        pk = lane ^ k
        return (jnp.where(m, a, b) + jnp.where(m, _shuf(a, pk),
                                               _shuf(b, pk)))

    def do_chunk(kvb, qb, eb, dstb, count):
        @plsc.parallel_loop(0, count, 1, unroll=2)
        def edge_body(e):
            t = [kvb[e, pl.ds(h * DH, DH)]
                 * qb[e, pl.ds(h * DH, DH)]
                 * eb[e, pl.ds(h * DH, DH)] for h in range(H)]
            m = [merge(t[2 * j], t[2 * j + 1], 1) for j in range(4)]
            n = [merge(m[2 * j], m[2 * j + 1], 2) for j in range(2)]
            p = merge(n[0], n[1], 4)
            sacc = p + _shuf(p, lane ^ 8)
            score = jnp.exp(jnp.clip(sacc * 0.25, -5.0, 5.0))
            msgbuf[e, pl.ds(D, 16)] = score
            for h in range(H):
                bc = _shuf(score, jnp.full((16,), h, jnp.int32))
                msgbuf[e, pl.ds(h * DH, DH)] = (
                    kvb[e, pl.ds(D + h * DH, DH)] * bc)
        if count == C:
            pltpu.sync_copy(msgbuf, acc.at[dstb], add=True)
        else:
            pltpu.sync_copy(msgbuf.at[pl.ds(0, count)], acc.at[dstb],
                            add=True)

    def load_idx(base, idxb):
        pltpu.sync_copy(eidx.at[:, pl.ds(base, idxb.shape[1])], idxb)

    def issue(base, idxb, kvb, qb, eb, sem):
        pltpu.async_copy(kvh.at[idxb.at[0]], kvb, sem)
        pltpu.async_copy(qh.at[idxb.at[1]], qb, sem)
        pltpu.async_copy(eh.at[pl.ds(base, kvb.shape[0])], eb, sem)

    def drain(kvb, qb, eb, sem):
        pltpu.make_async_copy(kvh.at[pl.ds(0, kvb.shape[0])], kvb,
                              sem).wait()
        pltpu.make_async_copy(qh.at[pl.ds(0, qb.shape[0])], qb, sem).wait()
        pltpu.make_async_copy(eh.at[pl.ds(0, eb.shape[0])], eb, sem).wait()

    load_idx(tile_base, idxA)
    issue(tile_base, idxA, kvA, qA, eA, gsemA)

    def pair_body(j, carry):
        i1 = 2 * j + 1
        base1 = tile_base + i1 * C
        load_idx(base1, idxB)
        issue(base1, idxB, kvB, qB, eB, gsemB)
        drain(kvA, qA, eA, gsemA)
        do_chunk(kvA, qA, eA, idxA.at[1], C)
        i2 = jnp.minimum(2 * j + 2, NF - 1)
        base2 = tile_base + i2 * C
        load_idx(base2, idxA)
        issue(base2, idxA, kvA, qA, eA, gsemA)
        drain(kvB, qB, eB, gsemB)
        do_chunk(kvB, qB, eB, idxB.at[1], C)
        return carry
    lax.fori_loop(0, NF // 2, pair_body, 0)

    drain(kvA, qA, eA, gsemA)

    tbase = tile_base + NF * C
    pltpu.sync_copy(eidx.at[:, pl.ds(tbase, CT)], idxT)
    pltpu.async_copy(kvh.at[idxT.at[0]], kvA.at[pl.ds(0, CT)], gsemA)
    pltpu.async_copy(qh.at[idxT.at[1]], qA.at[pl.ds(0, CT)], gsemA)
    pltpu.async_copy(eh.at[pl.ds(tbase, CT)], eA.at[pl.ds(0, CT)], gsemA)
    pltpu.make_async_copy(kvh.at[pl.ds(0, CT)], kvA.at[pl.ds(0, CT)],
                          gsemA).wait()
    pltpu.make_async_copy(qh.at[pl.ds(0, CT)], qA.at[pl.ds(0, CT)],
                          gsemA).wait()
    pltpu.make_async_copy(eh.at[pl.ds(0, CT)], eA.at[pl.ds(0, CT)],
                          gsemA).wait()
    do_chunk(kvA, qA, eA, idxT.at[1], CT)

    plsc.subcore_barrier()
    out_base = c * NPAD + row0
    pltpu.sync_copy(acc.at[pl.ds(row0, ROWS_PER_TILE)],
                    out.at[pl.ds(out_base, ROWS_PER_TILE)])


def _combine_body(p0_ref, p1_ref, o_ref):
    a = p0_ref[0] + p1_ref[0]
    for h in range(H):
        wv = a[:, h * DH:(h + 1) * DH]
        z = a[:, D + h:D + h + 1]
        o_ref[:, h * DH:(h + 1) * DH] = wv / (z + 1e-6)


def _combine(partials):
    blk = 1000
    grid = (N // blk,)
    p3 = partials.reshape(NC, NPAD, W)
    return pl.pallas_call(
        _combine_body,
        grid=grid,
        in_specs=[pl.BlockSpec((1, blk, W), lambda i: (0, i, 0)),
                  pl.BlockSpec((1, blk, W), lambda i: (1, i, 0))],
        out_specs=pl.BlockSpec((blk, D), lambda i: (i, 0)),
        out_shape=jax.ShapeDtypeStruct((N, D), jnp.float32),
    )(p3, p3)


def kernel(x, edge_index, edge_attr, WQ, WK, WE, WV):
    qh, kvh = _qkv(x, WQ, WK, WV)
    eh = _eproj(edge_attr, WE)
    partials = _sc_attn(kvh, qh, eh, edge_index)
    return _combine(partials)


# speedup vs baseline: 1.3030x; 1.0672x over previous
import functools

import jax
import jax.numpy as jnp
from jax import lax
from jax.experimental import pallas as pl
from jax.experimental.pallas import tpu as pltpu
from jax.experimental.pallas import tpu_sc as plsc

N = 10000
E = 320000
D = 128
DE = 16
H = 8
DH = 16

NC = 2
NS = 16
NW = NC * NS
PER_TILE = E // NW
C = 32
W = 144
NPAD = 10240
ROWS_PER_TILE = NPAD // NS


def _qkv_body(x_ref, wq_ref, wk_ref, wv_ref, q_ref, kv_ref):
    xb = x_ref[...]
    q_ref[...] = jnp.dot(xb, wq_ref[...], preferred_element_type=jnp.float32)
    kv_ref[:, :D] = jnp.dot(xb, wk_ref[...], preferred_element_type=jnp.float32)
    kv_ref[:, D:] = jnp.dot(xb, wv_ref[...], preferred_element_type=jnp.float32)


def _qkv(x, WQ, WK, WV):
    blk = 1000
    grid = (N // blk,)
    spec_x = pl.BlockSpec((blk, D), lambda i: (i, 0))
    spec_w = pl.BlockSpec((D, D), lambda i: (0, 0))
    return pl.pallas_call(
        _qkv_body,
        grid=grid,
        in_specs=[spec_x, spec_w, spec_w, spec_w],
        out_specs=[pl.BlockSpec((blk, D), lambda i: (i, 0)),
                   pl.BlockSpec((blk, 2 * D), lambda i: (i, 0))],
        out_shape=[jax.ShapeDtypeStruct((N, D), jnp.float32),
                   jax.ShapeDtypeStruct((N, 2 * D), jnp.float32)],
    )(x, WQ, WK, WV)


def _eproj_body(ea_ref, we_ref, eh_ref):
    eh_ref[...] = jnp.dot(ea_ref[...], we_ref[...],
                          preferred_element_type=jnp.float32)


def _eproj(edge_attr, WE):
    blk = 4000
    grid = (E // blk,)
    return pl.pallas_call(
        _eproj_body,
        grid=grid,
        in_specs=[pl.BlockSpec((blk, DE), lambda i: (i, 0)),
                  pl.BlockSpec((DE, D), lambda i: (0, 0))],
        out_specs=pl.BlockSpec((blk, D), lambda i: (i, 0)),
        out_shape=jax.ShapeDtypeStruct((E, D), jnp.float32),
    )(edge_attr, WE)


_mesh = plsc.VectorSubcoreMesh(core_axis_name="c", subcore_axis_name="s")

_GDN = lax.GatherDimensionNumbers(
    offset_dims=(), collapsed_slice_dims=(0,), start_index_map=(0,))


def _shuf(v, perm):
    return lax.gather(v, perm[:, None], _GDN, (1,),
                      mode=lax.GatherScatterMode.PROMISE_IN_BOUNDS)


NF = PER_TILE // C
CT = PER_TILE - NF * C


@functools.partial(
    pl.kernel,
    out_type=jax.ShapeDtypeStruct((NC * NPAD, W), jnp.float32),
    mesh=_mesh,
    scratch_types=[
        pltpu.VMEM((2, C), jnp.int32),
        pltpu.VMEM((2, C), jnp.int32),
        pltpu.VMEM((2, C), jnp.int32),
        pltpu.VMEM((2, C), jnp.int32),
        pltpu.VMEM((2, CT), jnp.int32),
        pltpu.VMEM((C, 2 * D), jnp.float32),
        pltpu.VMEM((C, 2 * D), jnp.float32),
        pltpu.VMEM((C, D), jnp.float32),
        pltpu.VMEM((C, D), jnp.float32),
        pltpu.VMEM((C, D), jnp.float32),
        pltpu.VMEM((C, D), jnp.float32),
        pltpu.VMEM((C, W), jnp.float32),
        pltpu.VMEM_SHARED((NPAD, W), jnp.float32),
        pltpu.SemaphoreType.DMA,
        pltpu.SemaphoreType.DMA,
        pltpu.SemaphoreType.DMA,
        pltpu.SemaphoreType.DMA,
        pltpu.SemaphoreType.DMA,
        pltpu.SemaphoreType.DMA,
    ],
    compiler_params=pltpu.CompilerParams(use_tc_tiling_on_sc=False),
)
def _sc_attn(kvh, qh, eh, eidx, out,
             idx0, idx1, idx2, idx3, idxT,
             kvA, kvB, qA, qB, eA, eB, msgbuf, acc,
             gsemA, gsemB, isem0, isem1, isem2, isem3):
    c = lax.axis_index("c")
    s = lax.axis_index("s")
    lane = lax.iota(jnp.int32, 16)
    zero16 = jnp.zeros((16,), jnp.float32)

    def zrow(r, carry):
        for j in range(W // 16):
            msgbuf[r, pl.ds(j * 16, 16)] = zero16
        return carry
    lax.fori_loop(0, C, zrow, 0)
    row0 = s * ROWS_PER_TILE
    for i in range(ROWS_PER_TILE // C):
        pltpu.sync_copy(msgbuf, acc.at[pl.ds(row0 + i * C, C)])
    plsc.subcore_barrier()

    tile_base = (c * NS + s) * PER_TILE

    def merge(a, b, k):
        m = (lane & k) == 0
        pk = lane ^ k
        return (jnp.where(m, a, b) + jnp.where(m, _shuf(a, pk),
                                               _shuf(b, pk)))

    def do_chunk(kvb, qb, eb, dstb, count):
        @plsc.parallel_loop(0, count, 1, unroll=2)
        def edge_body(e):
            t = [kvb[e, pl.ds(h * DH, DH)]
                 * qb[e, pl.ds(h * DH, DH)]
                 * eb[e, pl.ds(h * DH, DH)] for h in range(H)]
            m = [merge(t[2 * j], t[2 * j + 1], 1) for j in range(4)]
            n = [merge(m[2 * j], m[2 * j + 1], 2) for j in range(2)]
            p = merge(n[0], n[1], 4)
            sacc = p + _shuf(p, lane ^ 8)
            score = jnp.exp(jnp.clip(sacc * 0.25, -5.0, 5.0))
            msgbuf[e, pl.ds(D, 16)] = score
            for h in range(H):
                bc = _shuf(score, jnp.full((16,), h, jnp.int32))
                msgbuf[e, pl.ds(h * DH, DH)] = (
                    kvb[e, pl.ds(D + h * DH, DH)] * bc)
        if count == C:
            pltpu.sync_copy(msgbuf, acc.at[dstb], add=True)
        else:
            pltpu.sync_copy(msgbuf.at[pl.ds(0, count)], acc.at[dstb],
                            add=True)

    def load_idx(base, idxb):
        pltpu.sync_copy(eidx.at[:, pl.ds(base, idxb.shape[1])], idxb)

    def issue(base, idxb, kvb, qb, eb, sem):
        pltpu.async_copy(kvh.at[idxb.at[0]], kvb, sem)
        pltpu.async_copy(qh.at[idxb.at[1]], qb, sem)
        pltpu.async_copy(eh.at[pl.ds(base, kvb.shape[0])], eb, sem)

    def drain(kvb, qb, eb, sem):
        pltpu.make_async_copy(kvh.at[pl.ds(0, kvb.shape[0])], kvb,
                              sem).wait()
        pltpu.make_async_copy(qh.at[pl.ds(0, qb.shape[0])], qb, sem).wait()
        pltpu.make_async_copy(eh.at[pl.ds(0, eb.shape[0])], eb, sem).wait()

    iring = [idx0, idx1, idx2, idx3]
    isems = [isem0, isem1, isem2, isem3]
    data = [(kvA, qA, eA, gsemA), (kvB, qB, eB, gsemB)]

    def idx_async(base, r):
        pltpu.async_copy(eidx.at[:, pl.ds(base, C)], iring[r], isems[r])

    def idx_wait(r):
        pltpu.make_async_copy(eidx.at[:, pl.ds(0, C)], iring[r],
                              isems[r]).wait()

    pltpu.sync_copy(eidx.at[:, pl.ds(tile_base, C)], idx0)
    for r in (1, 2, 3):
        idx_async(tile_base + r * C, r)
    issue(tile_base, idx0, kvA, qA, eA, gsemA)

    def quad_body(j, carry):
        i0 = 4 * j
        for k in range(4):
            i = i0 + k
            ip1 = jnp.minimum(i + 1, NF - 1)
            idx_wait((k + 1) % 4)
            kvb, qb, eb, sem = data[(k + 1) % 2]
            issue(tile_base + ip1 * C, iring[(k + 1) % 4], kvb, qb, eb, sem)
            kvb, qb, eb, sem = data[k % 2]
            drain(kvb, qb, eb, sem)
            do_chunk(kvb, qb, eb, iring[k].at[1], C)
            ip4 = jnp.minimum(i + 4, NF - 1)
            idx_async(tile_base + ip4 * C, k)
        return carry
    lax.fori_loop(0, NF // 4, quad_body, 0)

    for r in (1, 2, 3):
        idx_wait(r)
    drain(kvA, qA, eA, gsemA)

    tbase = tile_base + NF * C
    pltpu.sync_copy(eidx.at[:, pl.ds(tbase, CT)], idxT)
    pltpu.async_copy(kvh.at[idxT.at[0]], kvA.at[pl.ds(0, CT)], gsemA)
    pltpu.async_copy(qh.at[idxT.at[1]], qA.at[pl.ds(0, CT)], gsemA)
    pltpu.async_copy(eh.at[pl.ds(tbase, CT)], eA.at[pl.ds(0, CT)], gsemA)
    pltpu.make_async_copy(kvh.at[pl.ds(0, CT)], kvA.at[pl.ds(0, CT)],
                          gsemA).wait()
    pltpu.make_async_copy(qh.at[pl.ds(0, CT)], qA.at[pl.ds(0, CT)],
                          gsemA).wait()
    pltpu.make_async_copy(eh.at[pl.ds(0, CT)], eA.at[pl.ds(0, CT)],
                          gsemA).wait()
    do_chunk(kvA, qA, eA, idxT.at[1], CT)

    plsc.subcore_barrier()
    out_base = c * NPAD + row0
    pltpu.sync_copy(acc.at[pl.ds(row0, ROWS_PER_TILE)],
                    out.at[pl.ds(out_base, ROWS_PER_TILE)])


def _combine_body(p0_ref, p1_ref, o_ref):
    a = p0_ref[0] + p1_ref[0]
    for h in range(H):
        wv = a[:, h * DH:(h + 1) * DH]
        z = a[:, D + h:D + h + 1]
        o_ref[:, h * DH:(h + 1) * DH] = wv / (z + 1e-6)


def _combine(partials):
    blk = 1000
    grid = (N // blk,)
    p3 = partials.reshape(NC, NPAD, W)
    return pl.pallas_call(
        _combine_body,
        grid=grid,
        in_specs=[pl.BlockSpec((1, blk, W), lambda i: (0, i, 0)),
                  pl.BlockSpec((1, blk, W), lambda i: (1, i, 0))],
        out_specs=pl.BlockSpec((blk, D), lambda i: (i, 0)),
        out_shape=jax.ShapeDtypeStruct((N, D), jnp.float32),
    )(p3, p3)


def kernel(x, edge_index, edge_attr, WQ, WK, WE, WV):
    qh, kvh = _qkv(x, WQ, WK, WV)
    eh = _eproj(edge_attr, WE)
    partials = _sc_attn(kvh, qh, eh, edge_index)
    return _combine(partials)


# speedup vs baseline: 1.3043x; 1.0010x over previous
import functools

import jax
import jax.numpy as jnp
from jax import lax
from jax.experimental import pallas as pl
from jax.experimental.pallas import tpu as pltpu
from jax.experimental.pallas import tpu_sc as plsc

N = 10000
E = 320000
D = 128
DE = 16
H = 8
DH = 16

NC = 2
NS = 16
NW = NC * NS
PER_TILE = E // NW
C = 32
W = 144
NPAD = 10240
ROWS_PER_TILE = NPAD // NS


def _qkv_body(x_ref, wq_ref, wk_ref, wv_ref, q_ref, kv_ref):
    xb = x_ref[...]
    q_ref[...] = jnp.dot(xb, wq_ref[...], preferred_element_type=jnp.float32)
    kv_ref[:, :D] = jnp.dot(xb, wk_ref[...], preferred_element_type=jnp.float32)
    kv_ref[:, D:] = jnp.dot(xb, wv_ref[...], preferred_element_type=jnp.float32)


def _qkv(x, WQ, WK, WV):
    blk = 1000
    grid = (N // blk,)
    spec_x = pl.BlockSpec((blk, D), lambda i: (i, 0))
    spec_w = pl.BlockSpec((D, D), lambda i: (0, 0))
    return pl.pallas_call(
        _qkv_body,
        grid=grid,
        in_specs=[spec_x, spec_w, spec_w, spec_w],
        out_specs=[pl.BlockSpec((blk, D), lambda i: (i, 0)),
                   pl.BlockSpec((blk, 2 * D), lambda i: (i, 0))],
        out_shape=[jax.ShapeDtypeStruct((N, D), jnp.float32),
                   jax.ShapeDtypeStruct((N, 2 * D), jnp.float32)],
    )(x, WQ, WK, WV)


def _eproj_body(ea_ref, we_ref, eh_ref):
    eh_ref[...] = jnp.dot(ea_ref[...], we_ref[...],
                          preferred_element_type=jnp.float32)


def _eproj(edge_attr, WE):
    blk = 4000
    grid = (E // blk,)
    return pl.pallas_call(
        _eproj_body,
        grid=grid,
        in_specs=[pl.BlockSpec((blk, DE), lambda i: (i, 0)),
                  pl.BlockSpec((DE, D), lambda i: (0, 0))],
        out_specs=pl.BlockSpec((blk, D), lambda i: (i, 0)),
        out_shape=jax.ShapeDtypeStruct((E, D), jnp.float32),
    )(edge_attr, WE)


_mesh = plsc.VectorSubcoreMesh(core_axis_name="c", subcore_axis_name="s")

_GDN = lax.GatherDimensionNumbers(
    offset_dims=(), collapsed_slice_dims=(0,), start_index_map=(0,))


def _shuf(v, perm):
    return lax.gather(v, perm[:, None], _GDN, (1,),
                      mode=lax.GatherScatterMode.PROMISE_IN_BOUNDS)


NF = PER_TILE // C
CT = PER_TILE - NF * C


@functools.partial(
    pl.kernel,
    out_type=jax.ShapeDtypeStruct((NC, NPAD, W), jnp.float32),
    mesh=_mesh,
    scratch_types=[
        pltpu.VMEM((2, C), jnp.int32),
        pltpu.VMEM((2, C), jnp.int32),
        pltpu.VMEM((2, C), jnp.int32),
        pltpu.VMEM((2, C), jnp.int32),
        pltpu.VMEM((2, CT), jnp.int32),
        pltpu.VMEM((C, 2 * D), jnp.float32),
        pltpu.VMEM((C, 2 * D), jnp.float32),
        pltpu.VMEM((C, D), jnp.float32),
        pltpu.VMEM((C, D), jnp.float32),
        pltpu.VMEM((C, D), jnp.float32),
        pltpu.VMEM((C, D), jnp.float32),
        pltpu.VMEM((C, W), jnp.float32),
        pltpu.VMEM_SHARED((NPAD, W), jnp.float32),
        pltpu.SemaphoreType.DMA,
        pltpu.SemaphoreType.DMA,
        pltpu.SemaphoreType.DMA,
        pltpu.SemaphoreType.DMA,
        pltpu.SemaphoreType.DMA,
        pltpu.SemaphoreType.DMA,
    ],
    compiler_params=pltpu.CompilerParams(use_tc_tiling_on_sc=False),
)
def _sc_attn(kvh, qh, eh, eidx, out,
             idx0, idx1, idx2, idx3, idxT,
             kvA, kvB, qA, qB, eA, eB, msgbuf, acc,
             gsemA, gsemB, isem0, isem1, isem2, isem3):
    c = lax.axis_index("c")
    s = lax.axis_index("s")
    lane = lax.iota(jnp.int32, 16)
    zero16 = jnp.zeros((16,), jnp.float32)

    def zrow(r, carry):
        for j in range(W // 16):
            msgbuf[r, pl.ds(j * 16, 16)] = zero16
        return carry
    lax.fori_loop(0, C, zrow, 0)
    row0 = s * ROWS_PER_TILE
    for i in range(ROWS_PER_TILE // C):
        pltpu.sync_copy(msgbuf, acc.at[pl.ds(row0 + i * C, C)])
    plsc.subcore_barrier()

    tile_base = (c * NS + s) * PER_TILE

    def merge(a, b, k):
        m = (lane & k) == 0
        pk = lane ^ k
        return (jnp.where(m, a, b) + jnp.where(m, _shuf(a, pk),
                                               _shuf(b, pk)))

    def do_chunk(kvb, qb, eb, dstb, count):
        @plsc.parallel_loop(0, count, 1, unroll=2)
        def edge_body(e):
            t = [kvb[e, pl.ds(h * DH, DH)]
                 * qb[e, pl.ds(h * DH, DH)]
                 * eb[e, pl.ds(h * DH, DH)] for h in range(H)]
            m = [merge(t[2 * j], t[2 * j + 1], 1) for j in range(4)]
            n = [merge(m[2 * j], m[2 * j + 1], 2) for j in range(2)]
            p = merge(n[0], n[1], 4)
            sacc = p + _shuf(p, lane ^ 8)
            score = jnp.exp(jnp.clip(sacc * 0.25, -5.0, 5.0))
            msgbuf[e, pl.ds(D, 16)] = score
            for h in range(H):
                bc = _shuf(score, jnp.full((16,), h, jnp.int32))
                msgbuf[e, pl.ds(h * DH, DH)] = (
                    kvb[e, pl.ds(D + h * DH, DH)] * bc)
        if count == C:
            pltpu.sync_copy(msgbuf, acc.at[dstb], add=True)
        else:
            pltpu.sync_copy(msgbuf.at[pl.ds(0, count)], acc.at[dstb],
                            add=True)

    def load_idx(base, idxb):
        pltpu.sync_copy(eidx.at[:, pl.ds(base, idxb.shape[1])], idxb)

    def issue(base, idxb, kvb, qb, eb, sem):
        pltpu.async_copy(kvh.at[idxb.at[0]], kvb, sem)
        pltpu.async_copy(qh.at[idxb.at[1]], qb, sem)
        pltpu.async_copy(eh.at[pl.ds(base, kvb.shape[0])], eb, sem)

    def drain(kvb, qb, eb, sem):
        pltpu.make_async_copy(kvh.at[pl.ds(0, kvb.shape[0])], kvb,
                              sem).wait()
        pltpu.make_async_copy(qh.at[pl.ds(0, qb.shape[0])], qb, sem).wait()
        pltpu.make_async_copy(eh.at[pl.ds(0, eb.shape[0])], eb, sem).wait()

    iring = [idx0, idx1, idx2, idx3]
    isems = [isem0, isem1, isem2, isem3]
    data = [(kvA, qA, eA, gsemA), (kvB, qB, eB, gsemB)]

    def idx_async(base, r):
        pltpu.async_copy(eidx.at[:, pl.ds(base, C)], iring[r], isems[r])

    def idx_wait(r):
        pltpu.make_async_copy(eidx.at[:, pl.ds(0, C)], iring[r],
                              isems[r]).wait()

    pltpu.sync_copy(eidx.at[:, pl.ds(tile_base, C)], idx0)
    for r in (1, 2, 3):
        idx_async(tile_base + r * C, r)
    issue(tile_base, idx0, kvA, qA, eA, gsemA)

    def quad_body(j, carry):
        i0 = 4 * j
        for k in range(4):
            i = i0 + k
            ip1 = jnp.minimum(i + 1, NF - 1)
            idx_wait((k + 1) % 4)
            kvb, qb, eb, sem = data[(k + 1) % 2]
            issue(tile_base + ip1 * C, iring[(k + 1) % 4], kvb, qb, eb, sem)
            kvb, qb, eb, sem = data[k % 2]
            drain(kvb, qb, eb, sem)
            do_chunk(kvb, qb, eb, iring[k].at[1], C)
            ip4 = jnp.minimum(i + 4, NF - 1)
            idx_async(tile_base + ip4 * C, k)
        return carry
    lax.fori_loop(0, NF // 4, quad_body, 0)

    for r in (1, 2, 3):
        idx_wait(r)
    drain(kvA, qA, eA, gsemA)

    tbase = tile_base + NF * C
    pltpu.sync_copy(eidx.at[:, pl.ds(tbase, CT)], idxT)
    pltpu.async_copy(kvh.at[idxT.at[0]], kvA.at[pl.ds(0, CT)], gsemA)
    pltpu.async_copy(qh.at[idxT.at[1]], qA.at[pl.ds(0, CT)], gsemA)
    pltpu.async_copy(eh.at[pl.ds(tbase, CT)], eA.at[pl.ds(0, CT)], gsemA)
    pltpu.make_async_copy(kvh.at[pl.ds(0, CT)], kvA.at[pl.ds(0, CT)],
                          gsemA).wait()
    pltpu.make_async_copy(qh.at[pl.ds(0, CT)], qA.at[pl.ds(0, CT)],
                          gsemA).wait()
    pltpu.make_async_copy(eh.at[pl.ds(0, CT)], eA.at[pl.ds(0, CT)],
                          gsemA).wait()
    do_chunk(kvA, qA, eA, idxT.at[1], CT)

    plsc.subcore_barrier()
    pltpu.sync_copy(acc.at[pl.ds(row0, ROWS_PER_TILE)],
                    out.at[c, pl.ds(row0, ROWS_PER_TILE)])


def _combine_body(p0_ref, p1_ref, o_ref):
    a = p0_ref[0] + p1_ref[0]
    for h in range(H):
        wv = a[:, h * DH:(h + 1) * DH]
        z = a[:, D + h:D + h + 1]
        o_ref[:, h * DH:(h + 1) * DH] = wv / (z + 1e-6)


def _combine(partials):
    blk = 1000
    grid = (N // blk,)
    p3 = partials
    return pl.pallas_call(
        _combine_body,
        grid=grid,
        in_specs=[pl.BlockSpec((1, blk, W), lambda i: (0, i, 0)),
                  pl.BlockSpec((1, blk, W), lambda i: (1, i, 0))],
        out_specs=pl.BlockSpec((blk, D), lambda i: (i, 0)),
        out_shape=jax.ShapeDtypeStruct((N, D), jnp.float32),
    )(p3, p3)


def kernel(x, edge_index, edge_attr, WQ, WK, WE, WV):
    qh, kvh = _qkv(x, WQ, WK, WV)
    eh = _eproj(edge_attr, WE)
    partials = _sc_attn(kvh, qh, eh, edge_index)
    return _combine(partials)
